# baseline (speedup 1.0000x reference)
import functools

import jax
import jax.numpy as jnp
from jax import lax
from jax.experimental import pallas as pl
from jax.experimental.pallas import tpu as pltpu
from jax.experimental.pallas import tpu_sc as plsc

VOCAB = 50265
HIDDEN = 1024
PAD_IDX = 1
MAX_POS = 4098
LN_EPS = 1e-12
B, S = 4, 2048
TOK = B * S

NC = 2
NS = 16
NW = NC * NS
L = 16

P = 4
TOKS = TOK // P
BPW = TOKS // NW
CH = 16
NCH = BPW // CH
CPB = 4

_sc_mesh = plsc.VectorSubcoreMesh(core_axis_name="c", subcore_axis_name="s")


def _pid_body(ids_ref, pid_ref):
    x = ids_ref[...]
    m = (x != PAD_IDX).astype(jnp.int32)
    cs = m
    sh = 1
    while sh < S:
        cs = cs + jnp.concatenate(
            [jnp.zeros((B, sh), jnp.int32), cs[:, :S - sh]], axis=1)
        sh *= 2
    pid_ref[...] = cs * m + PAD_IDX


def _position_ids_tc(input_ids):
    return pl.pallas_call(
        _pid_body,
        out_shape=jax.ShapeDtypeStruct((B, S), jnp.int32),
    )(input_ids)


@functools.partial(
    pl.kernel,
    out_type=jax.ShapeDtypeStruct((TOKS, HIDDEN), jnp.float32),
    mesh=_sc_mesh,
    scratch_types=(
        [pltpu.VMEM((BPW,), jnp.int32),
         pltpu.VMEM((BPW,), jnp.int32)]
        + [pltpu.VMEM((CH, HIDDEN), jnp.float32)] * 4
        + [pltpu.VMEM((CH, HIDDEN), jnp.float32)] * 3
        + [pltpu.SemaphoreType.DMA] * 11
    ),
)
def _sc_embed(ids_hbm, pid_hbm, word_hbm, pos_hbm, out_hbm,
              ids_v, pid_v, w0, w1, w2, w3, p0, p1, p2, *sems):
    wid = lax.axis_index("s") * NC + lax.axis_index("c")
    base = wid * BPW

    pltpu.sync_copy(ids_hbm.at[pl.ds(base, BPW)], ids_v)
    pltpu.sync_copy(pid_hbm.at[pl.ds(base, BPW)], pid_v)

    wbufs = (w0, w1, w2, w3)
    pbufs = (p0, p1, p2)
    wsems = sems[0:4]
    psems = sems[4:7]
    osems = sems[7:11]

    def wgather(c, u):
        return pltpu.async_copy(
            word_hbm.at[ids_v.at[pl.ds(c * CH, CH)]], wbufs[u], wsems[u])

    def pgather(c, q):
        return pltpu.async_copy(
            pos_hbm.at[pid_v.at[pl.ds(c * CH, CH)]], pbufs[q], psems[q])

    def add_chunk(wb, pb):
        def row_body(j, _):
            for k in range(HIDDEN // L):
                sl = pl.ds(k * L, L)
                plsc.addupdate(wb.at[j, sl], pb[j, sl])
            return 0
        lax.fori_loop(0, CH, row_body, 0)

    def body(t, _):
        c0 = t * CPB
        wg = [wgather(c0 + u, u) for u in range(CPB)]
        pg = [pgather(c0 + q, q) for q in range(3)]
        outs = []
        for u in range(CPB):
            wg[u].wait()
            pg[u].wait()
            add_chunk(wbufs[u], pbufs[u % 3])
            outs.append(pltpu.async_copy(
                wbufs[u], out_hbm.at[pl.ds(base + (c0 + u) * CH, CH)],
                osems[u]))
            if u == 0:
                pg.append(pgather(c0 + 3, 0))
        for o in outs:
            o.wait()
        return 0

    lax.fori_loop(0, NCH // CPB, body, 0)


def _ln_first_body(x_ref, t_ref, g_ref, b_ref, o_ref):
    _ln_compute(x_ref, t_ref, g_ref, b_ref, o_ref)


def _ln_acc_body(acc_ref, x_ref, t_ref, g_ref, b_ref, o_ref):
    del acc_ref
    _ln_compute(x_ref, t_ref, g_ref, b_ref, o_ref)


def _ln_compute(x_ref, t_ref, g_ref, b_ref, o_ref):
    x = x_ref[...] + t_ref[...]
    mu = jnp.mean(x, axis=-1, keepdims=True)
    m2 = jnp.mean(x * x, axis=-1, keepdims=True)
    var = m2 - mu * mu
    o_ref[...] = (x - mu) * lax.rsqrt(var + LN_EPS) * g_ref[...] + b_ref[...]


_LN_BLK = 1024
_NBLK = TOKS // _LN_BLK


def _layernorm_slice(p, summ, acc, type_row, gamma, beta):
    grid = (_NBLK,)
    xspec = pl.BlockSpec((_LN_BLK, HIDDEN), lambda i: (i, 0))
    rspec = pl.BlockSpec((1, HIDDEN), lambda i: (0, 0))
    ospec = pl.BlockSpec((_LN_BLK, HIDDEN), lambda i, _p=p: (_p * _NBLK + i, 0))
    out_shape = jax.ShapeDtypeStruct((TOK, HIDDEN), jnp.float32)
    if acc is None:
        return pl.pallas_call(
            _ln_first_body,
            grid=grid,
            in_specs=[xspec, rspec, rspec, rspec],
            out_specs=ospec,
            out_shape=out_shape,
        )(summ, type_row, gamma, beta)
    return pl.pallas_call(
        _ln_acc_body,
        grid=grid,
        in_specs=[pl.BlockSpec(memory_space=pltpu.MemorySpace.HBM),
                  xspec, rspec, rspec, rspec],
        out_specs=ospec,
        out_shape=out_shape,
        input_output_aliases={0: 0},
    )(acc, summ, type_row, gamma, beta)


def kernel(input_ids, word_emb, pos_emb, type_emb, ln_gamma, ln_beta):
    pid = _position_ids_tc(input_ids.astype(jnp.int32))
    ids_flat = input_ids.reshape(TOK).astype(jnp.int32)
    pid_flat = pid.reshape(TOK)
    type_row = type_emb[0:1]
    gamma = ln_gamma.reshape(1, HIDDEN)
    beta = ln_beta.reshape(1, HIDDEN)

    summs = []
    for p in range(P):
        sl = slice(p * TOKS, (p + 1) * TOKS)
        summs.append(
            _sc_embed(ids_flat[sl], pid_flat[sl], word_emb, pos_emb))
    emb = None
    for p in range(P):
        emb = _layernorm_slice(p, summs[p], emb, type_row, gamma, beta)

    return (emb.reshape(B, S, HIDDEN),
            pid.reshape(B, S).astype(input_ids.dtype))


# speedup vs baseline: 1.0687x; 1.0687x over previous
import functools

import jax
import jax.numpy as jnp
from jax import lax
from jax.experimental import pallas as pl
from jax.experimental.pallas import tpu as pltpu
from jax.experimental.pallas import tpu_sc as plsc

VOCAB = 50265
HIDDEN = 1024
PAD_IDX = 1
MAX_POS = 4098
LN_EPS = 1e-12
B, S = 4, 2048
TOK = B * S

NC = 2
NS = 16
NW = NC * NS
L = 16

P = 1
TOKS = TOK // P
BPW = TOKS // NW
CH = 16
NCH = BPW // CH
CPB = 4

_sc_mesh = plsc.VectorSubcoreMesh(core_axis_name="c", subcore_axis_name="s")


def _pid_body(ids_ref, pid_ref):
    x = ids_ref[...]
    m = (x != PAD_IDX).astype(jnp.int32)
    cs = m
    sh = 1
    while sh < S:
        cs = cs + jnp.concatenate(
            [jnp.zeros((B, sh), jnp.int32), cs[:, :S - sh]], axis=1)
        sh *= 2
    pid_ref[...] = cs * m + PAD_IDX


def _position_ids_tc(input_ids):
    return pl.pallas_call(
        _pid_body,
        out_shape=jax.ShapeDtypeStruct((B, S), jnp.int32),
    )(input_ids)


@functools.partial(
    pl.kernel,
    out_type=jax.ShapeDtypeStruct((TOKS, HIDDEN), jnp.float32),
    mesh=_sc_mesh,
    scratch_types=(
        [pltpu.VMEM((BPW,), jnp.int32),
         pltpu.VMEM((BPW,), jnp.int32)]
        + [pltpu.VMEM((CH, HIDDEN), jnp.float32)] * 4
        + [pltpu.VMEM((CH, HIDDEN), jnp.float32)] * 3
        + [pltpu.SemaphoreType.DMA] * 11
    ),
)
def _sc_embed(ids_hbm, pid_hbm, word_hbm, pos_hbm, out_hbm,
              ids_v, pid_v, w0, w1, w2, w3, p0, p1, p2, *sems):
    wid = lax.axis_index("s") * NC + lax.axis_index("c")
    base = wid * BPW

    pltpu.sync_copy(ids_hbm.at[pl.ds(base, BPW)], ids_v)
    pltpu.sync_copy(pid_hbm.at[pl.ds(base, BPW)], pid_v)

    wbufs = (w0, w1, w2, w3)
    pbufs = (p0, p1, p2)
    wsems = sems[0:4]
    psems = sems[4:7]
    osems = sems[7:11]

    def wgather(c, u):
        return pltpu.async_copy(
            word_hbm.at[ids_v.at[pl.ds(c * CH, CH)]], wbufs[u], wsems[u])

    def pgather(c, q):
        return pltpu.async_copy(
            pos_hbm.at[pid_v.at[pl.ds(c * CH, CH)]], pbufs[q], psems[q])

    def add_chunk(wb, pb):
        def row_body(j, _):
            for k in range(HIDDEN // L):
                sl = pl.ds(k * L, L)
                plsc.addupdate(wb.at[j, sl], pb[j, sl])
            return 0
        lax.fori_loop(0, CH, row_body, 0)

    def out_wait(c, u):
        pltpu.make_async_copy(
            wbufs[u], out_hbm.at[pl.ds(base + c * CH, CH)],
            osems[u]).wait()

    def body(t, _):
        c0 = t * CPB
        wg = []
        for u in range(CPB):
            @pl.when(t > 0)
            def _(u=u):
                out_wait(c0 - CPB + u, u)

            wg.append(wgather(c0 + u, u))
        pg = [pgather(c0 + q, q) for q in range(3)]
        for u in range(CPB):
            wg[u].wait()
            pg[u].wait()
            add_chunk(wbufs[u], pbufs[u % 3])
            pltpu.async_copy(
                wbufs[u], out_hbm.at[pl.ds(base + (c0 + u) * CH, CH)],
                osems[u])
            if u == 0:
                pg.append(pgather(c0 + 3, 0))
        return 0

    lax.fori_loop(0, NCH // CPB, body, 0)
    for u in range(CPB):
        out_wait(NCH - CPB + u, u)


def _ln_first_body(x_ref, t_ref, g_ref, b_ref, o_ref):
    _ln_compute(x_ref, t_ref, g_ref, b_ref, o_ref)


def _ln_acc_body(acc_ref, x_ref, t_ref, g_ref, b_ref, o_ref):
    del acc_ref
    _ln_compute(x_ref, t_ref, g_ref, b_ref, o_ref)


def _ln_compute(x_ref, t_ref, g_ref, b_ref, o_ref):
    x = x_ref[...] + t_ref[...]
    mu = jnp.mean(x, axis=-1, keepdims=True)
    m2 = jnp.mean(x * x, axis=-1, keepdims=True)
    var = m2 - mu * mu
    o_ref[...] = (x - mu) * lax.rsqrt(var + LN_EPS) * g_ref[...] + b_ref[...]


_LN_BLK = 1024
_NBLK = TOKS // _LN_BLK


def _layernorm_slice(p, summ, acc, type_row, gamma, beta):
    grid = (_NBLK,)
    xspec = pl.BlockSpec((_LN_BLK, HIDDEN), lambda i: (i, 0))
    rspec = pl.BlockSpec((1, HIDDEN), lambda i: (0, 0))
    ospec = pl.BlockSpec((_LN_BLK, HIDDEN), lambda i, _p=p: (_p * _NBLK + i, 0))
    out_shape = jax.ShapeDtypeStruct((TOK, HIDDEN), jnp.float32)
    if acc is None:
        return pl.pallas_call(
            _ln_first_body,
            grid=grid,
            in_specs=[xspec, rspec, rspec, rspec],
            out_specs=ospec,
            out_shape=out_shape,
        )(summ, type_row, gamma, beta)
    return pl.pallas_call(
        _ln_acc_body,
        grid=grid,
        in_specs=[pl.BlockSpec(memory_space=pltpu.MemorySpace.HBM),
                  xspec, rspec, rspec, rspec],
        out_specs=ospec,
        out_shape=out_shape,
        input_output_aliases={0: 0},
    )(acc, summ, type_row, gamma, beta)


def kernel(input_ids, word_emb, pos_emb, type_emb, ln_gamma, ln_beta):
    pid = _position_ids_tc(input_ids.astype(jnp.int32))
    ids_flat = input_ids.reshape(TOK).astype(jnp.int32)
    pid_flat = pid.reshape(TOK)
    type_row = type_emb[0:1]
    gamma = ln_gamma.reshape(1, HIDDEN)
    beta = ln_beta.reshape(1, HIDDEN)

    summs = []
    for p in range(P):
        sl = slice(p * TOKS, (p + 1) * TOKS)
        summs.append(
            _sc_embed(ids_flat[sl], pid_flat[sl], word_emb, pos_emb))
    emb = None
    for p in range(P):
        emb = _layernorm_slice(p, summs[p], emb, type_row, gamma, beta)

    return (emb.reshape(B, S, HIDDEN),
            pid.reshape(B, S).astype(input_ids.dtype))
